# Initial kernel scaffold; baseline (speedup 1.0000x reference)
import jax
import jax.numpy as jnp
from jax.experimental import pallas as pl


def kernel(x, edge_index, pos, max_radius, num_nodes, Ws1, Ws2, Wp1, Wp2, Wd1, Wd2):
    raise NotImplementedError("write your pallas kernel here")


# speedup vs baseline: 2.9949x; 2.9949x over previous
import functools

import numpy as np
import jax
import jax.numpy as jnp
from jax import lax
from jax.experimental import pallas as pl
from jax.experimental.pallas import tpu as pltpu
from jax.experimental.pallas import tpu_sc as plsc

N = 10000
E = 320000
E2 = 327680
NW = 32
EW = E2 // NW
CH = 1024
K = EW // CH
G = 48
SROW = 16
NSUB = 16
NROWS = N // NSUB

C1 = float(np.sqrt(3.0))
C2 = float(np.sqrt(15.0))
C3 = float(np.sqrt(5.0) / 2.0)
C4 = float(np.sqrt(15.0) / 2.0)
EMBS = float(1.14136 * np.exp(2.0) * np.sqrt(10.0))
FS = float(1.0 / np.sqrt(10.0))
FH = float(np.sqrt(2.0))
FO = 1.0 / 8.0
SC0 = float(1.0 / np.sqrt(32.0))


def _sel_matrix():
    S = np.zeros((162, 40), np.float32)
    k = 0
    S[0, k] = 1.0
    S[1, k] = 1.0
    k += 1
    for i in range(1, 4):
        for j in range(1, 4):
            c = (i * 9 + j) * 2
            S[c, k] = 1.0
            S[c + 1, k] = 1.0
            k += 1
    for i in range(4, 9):
        for j in range(4, 9):
            c = (i * 9 + j) * 2
            S[c, k] = 1.0
            S[c + 1, k] = 1.0
            k += 1
    return S


_SEL = _sel_matrix()


def _prep_body(x_ref, pos_ref, sel_ref, tbl_ref):
    t = jnp.dot(x_ref[...], sel_ref[...], preferred_element_type=jnp.float32)
    tbl_ref[...] = jnp.concatenate(
        [t, pos_ref[...], jnp.zeros((t.shape[0], 5), jnp.float32)], axis=1)


def _build_tbl(x, pos, sel):
    nb = 10
    bs = N // nb
    return pl.pallas_call(
        _prep_body,
        grid=(nb,),
        in_specs=[pl.BlockSpec((bs, 162), lambda i: (i, 0)),
                  pl.BlockSpec((bs, 3), lambda i: (i, 0)),
                  pl.BlockSpec((162, 40), lambda i: (0, 0))],
        out_specs=pl.BlockSpec((bs, G), lambda i: (i, 0)),
        out_shape=jax.ShapeDtypeStruct((N, G), jnp.float32),
    )(x, pos, sel)


def _gather_body(tbl_hbm, pos16_hbm, row_hbm, col_hbm, pc_hbm, g_hbm,
                 rowi, coli, bufG, bufB, sem):
    cid = lax.axis_index("c")
    sid = lax.axis_index("s")
    wid = sid * 2 + cid
    pltpu.sync_copy(row_hbm.at[wid], rowi)
    pltpu.sync_copy(col_hbm.at[wid], coli)
    base = wid * EW
    for k in range(K):
        idr = rowi.at[pl.ds(k * CH, CH)]
        idc = coli.at[pl.ds(k * CH, CH)]
        cg = pltpu.async_copy(tbl_hbm.at[idr], bufG, sem)
        cb = pltpu.async_copy(pos16_hbm.at[idc], bufB, sem)
        cg.wait()
        cb.wait()
        pltpu.sync_copy(bufB, pc_hbm.at[pl.ds(base + k * CH, CH)])
        pltpu.sync_copy(bufG, g_hbm.at[pl.ds(base + k * CH, CH)])


def _gather(tbl, pos16, row2, col2):
    call = pl.kernel(
        _gather_body,
        out_type=(jax.ShapeDtypeStruct((E2, 16), jnp.float32),
                  jax.ShapeDtypeStruct((E2, G), jnp.float32)),
        mesh=plsc.VectorSubcoreMesh(core_axis_name="c", subcore_axis_name="s"),
        scratch_types=[pltpu.VMEM((EW,), jnp.int32),
                       pltpu.VMEM((EW,), jnp.int32),
                       pltpu.VMEM((CH, G), jnp.float32),
                       pltpu.VMEM((CH, 16), jnp.float32),
                       pltpu.SemaphoreType.DMA],
        compiler_params=pltpu.CompilerParams(use_tc_tiling_on_sc=False),
    )
    return call(tbl, pos16, row2, col2)


BE = 2048


def _edge_body(vals_ref, pc_ref, g_ref, ws1_ref, ws2_ref, wp1_ref,
               wp2_ref, wd1_ref, wd2_ref, out_ref):
    pc = pc_ref[...]
    g = g_ref[...]
    vx = g[:, 40:41] - pc[:, 0:1]
    vy = g[:, 41:42] - pc[:, 1:2]
    vz = g[:, 42:43] - pc[:, 2:3]
    r2 = vx * vx + vy * vy + vz * vz + 1e-12
    length = jnp.sqrt(r2)
    inv = 1.0 / length
    ux = vx * inv
    uy = vy * inv
    uz = vz * inv
    sh1 = jnp.concatenate([ux, uy, uz], axis=1) * C1
    sh2 = jnp.concatenate([C2 * ux * uy, C2 * uy * uz,
                           C3 * (3.0 * uz * uz - 1.0), C2 * ux * uz,
                           C4 * (ux * ux - uy * uy)], axis=1)
    diff = (length - vals_ref[0:1, 0:10]) * vals_ref[0:1, 10:11]
    ap = diff + 1.0
    bp = 1.0 - diff
    sa = jnp.where(ap > 0.0, jnp.exp(-1.0 / jnp.where(ap > 0.0, ap, 1.0)), 0.0)
    sb = jnp.where(bp > 0.0, jnp.exp(-1.0 / jnp.where(bp > 0.0, bp, 1.0)), 0.0)
    emb = EMBS * sa * sb

    def fc(w1_ref, w2_ref, oscale):
        h = jnp.dot(emb, w1_ref[...], preferred_element_type=jnp.float32) * FS
        h = FH * jnp.maximum(h, 0.0)
        return jnp.dot(h, w2_ref[...],
                       preferred_element_type=jnp.float32) * (FO * oscale)

    ws = fc(ws1_ref, ws2_ref, SC0)
    wp = fc(wp1_ref, wp2_ref, SC0 / 3.0)
    wd = fc(wd1_ref, wd2_ref, SC0 / 5.0)
    out_s = g[:, 0:1] * ws
    ip0 = jnp.sum(g[:, 1:4] * sh1, axis=1, keepdims=True)
    ip1 = jnp.sum(g[:, 4:7] * sh1, axis=1, keepdims=True)
    ip2 = jnp.sum(g[:, 7:10] * sh1, axis=1, keepdims=True)
    out_p = ip0 * wp[:, 0:4] + ip1 * wp[:, 4:8] + ip2 * wp[:, 8:12]
    id0 = jnp.sum(g[:, 10:15] * sh2, axis=1, keepdims=True)
    id1 = jnp.sum(g[:, 15:20] * sh2, axis=1, keepdims=True)
    id2 = jnp.sum(g[:, 20:25] * sh2, axis=1, keepdims=True)
    id3 = jnp.sum(g[:, 25:30] * sh2, axis=1, keepdims=True)
    id4 = jnp.sum(g[:, 30:35] * sh2, axis=1, keepdims=True)
    out_d = (id0 * wd[:, 0:4] + id1 * wd[:, 4:8] + id2 * wd[:, 8:12]
             + id3 * wd[:, 12:16] + id4 * wd[:, 16:20])
    out_ref[...] = jnp.concatenate(
        [out_s, out_p, out_d, jnp.zeros_like(out_s)], axis=1)


def _edge(vals, pc, ge, Ws1, Ws2, Wp1, Wp2, Wd1, Wd2):
    nb = E2 // BE
    wspec = lambda shape: pl.BlockSpec(shape, lambda i: (0, 0))
    return pl.pallas_call(
        _edge_body,
        grid=(nb,),
        in_specs=[wspec((1, 16)),
                  pl.BlockSpec((BE, 16), lambda i: (i, 0)),
                  pl.BlockSpec((BE, G), lambda i: (i, 0)),
                  wspec((10, 64)), wspec((64, 4)),
                  wspec((10, 64)), wspec((64, 12)),
                  wspec((10, 64)), wspec((64, 20))],
        out_specs=pl.BlockSpec((BE, SROW), lambda i: (i, 0)),
        out_shape=jax.ShapeDtypeStruct((E2, SROW), jnp.float32),
    )(vals, pc, ge, Ws1, Ws2, Wp1, Wp2, Wd1, Wd2)


def _scatter_body(s_hbm, col_hbm, z_hbm, part_hbm, coli, stage, acc):
    cid = lax.axis_index("c")
    sid = lax.axis_index("s")
    wid = sid * 2 + cid
    pltpu.sync_copy(col_hbm.at[wid], coli)
    pltpu.sync_copy(z_hbm, acc.at[pl.ds(sid * NROWS, NROWS)])
    plsc.subcore_barrier()
    base = wid * EW
    for k in range(K):
        pltpu.sync_copy(s_hbm.at[pl.ds(base + k * CH, CH)], stage)
        pltpu.sync_copy(stage, acc.at[coli.at[k]], add=True)
    plsc.subcore_barrier()
    pltpu.sync_copy(acc.at[pl.ds(sid * NROWS, NROWS)],
                    part_hbm.at[cid, pl.ds(sid * NROWS, NROWS)])


def _scatter(s16, col3, zer):
    call = pl.kernel(
        _scatter_body,
        out_type=jax.ShapeDtypeStruct((2, N, SROW), jnp.float32),
        mesh=plsc.VectorSubcoreMesh(core_axis_name="c", subcore_axis_name="s"),
        scratch_types=[pltpu.VMEM((K, CH), jnp.int32),
                       pltpu.VMEM((CH, SROW), jnp.float32),
                       pltpu.VMEM_SHARED((N, SROW), jnp.float32)],
        compiler_params=pltpu.CompilerParams(use_tc_tiling_on_sc=False),
    )
    return call(s16, col3, zer)


def _combine_body(p_ref, o_ref):
    s = p_ref[0] + p_ref[1]
    o_ref[...] = s[:, 0:12]


def _combine(part):
    nb = 10
    bs = N // nb
    return pl.pallas_call(
        _combine_body,
        grid=(nb,),
        in_specs=[pl.BlockSpec((2, bs, SROW), lambda i: (0, i, 0))],
        out_specs=pl.BlockSpec((bs, 12), lambda i: (i, 0)),
        out_shape=jax.ShapeDtypeStruct((N, 12), jnp.float32),
    )(part)


def kernel(x, edge_index, pos, max_radius, num_nodes, Ws1, Ws2, Wp1, Wp2,
           Wd1, Wd2):
    padz = jnp.zeros((E2 - E,), edge_index.dtype)
    rowp = jnp.concatenate([edge_index[0], padz])
    colp = jnp.concatenate([edge_index[1], padz])
    row2 = rowp.reshape(NW, EW)
    col2 = colp.reshape(NW, EW)
    col3 = colp.reshape(NW, K, CH)
    pos16 = jnp.pad(pos, ((0, 0), (0, 13)))
    mr = jnp.asarray(max_radius, jnp.float32)
    idx = jnp.arange(1, 11, dtype=jnp.float32)
    vals = jnp.concatenate([idx * (mr / 11.0), (11.0 / mr)[None],
                            jnp.zeros((5,), jnp.float32)]).reshape(1, 16)
    tbl = _build_tbl(x, pos, jnp.asarray(_SEL))
    pc, ge = _gather(tbl, pos16, row2, col2)
    s16 = _edge(vals, pc, ge, Ws1, Ws2, Wp1, Wp2, Wd1, Wd2)
    zer = jnp.zeros((NROWS, SROW), jnp.float32)
    part = _scatter(s16, col3, zer)
    return _combine(part)


# speedup vs baseline: 9.3998x; 3.1386x over previous
import functools

import numpy as np
import jax
import jax.numpy as jnp
from jax import lax
from jax.experimental import pallas as pl
from jax.experimental.pallas import tpu as pltpu
from jax.experimental.pallas import tpu_sc as plsc

N = 10000
E = 320000
E2 = 327680
NW = 32
EW = E2 // NW
CH = 1024
K = EW // CH
G = 48
SROW = 16
NSUB = 16
NROWS = N // NSUB

C1 = float(np.sqrt(3.0))
C2 = float(np.sqrt(15.0))
C3 = float(np.sqrt(5.0) / 2.0)
C4 = float(np.sqrt(15.0) / 2.0)
EMBS = float(1.14136 * np.exp(2.0) * np.sqrt(10.0))
FS = float(1.0 / np.sqrt(10.0))
FH = float(np.sqrt(2.0))
FO = 1.0 / 8.0
SC0 = float(1.0 / np.sqrt(32.0))


def _sel_matrix():
    S = np.zeros((162, 40), np.float32)
    k = 0
    S[0, k] = 1.0
    S[1, k] = 1.0
    k += 1
    for i in range(1, 4):
        for j in range(1, 4):
            c = (i * 9 + j) * 2
            S[c, k] = 1.0
            S[c + 1, k] = 1.0
            k += 1
    for i in range(4, 9):
        for j in range(4, 9):
            c = (i * 9 + j) * 2
            S[c, k] = 1.0
            S[c + 1, k] = 1.0
            k += 1
    return S


_SEL = _sel_matrix()


def _prep_body(x_ref, pos_ref, sel_ref, tbl_ref):
    t = jnp.dot(x_ref[...], sel_ref[...], preferred_element_type=jnp.float32)
    tbl_ref[...] = jnp.concatenate(
        [t, pos_ref[...], jnp.zeros((t.shape[0], 5), jnp.float32)], axis=1)


def _build_tbl(x, pos, sel):
    nb = 10
    bs = N // nb
    return pl.pallas_call(
        _prep_body,
        grid=(nb,),
        in_specs=[pl.BlockSpec((bs, 162), lambda i: (i, 0)),
                  pl.BlockSpec((bs, 3), lambda i: (i, 0)),
                  pl.BlockSpec((162, 40), lambda i: (0, 0))],
        out_specs=pl.BlockSpec((bs, G), lambda i: (i, 0)),
        out_shape=jax.ShapeDtypeStruct((N, G), jnp.float32),
    )(x, pos, sel)


def _gather_body(tbl_hbm, pos16_hbm, row_hbm, col_hbm, pc_hbm, g_hbm,
                 rowi, coli, bufG, bufB, sem):
    cid = lax.axis_index("c")
    sid = lax.axis_index("s")
    wid = sid * 2 + cid
    pltpu.sync_copy(row_hbm.at[wid], rowi)
    pltpu.sync_copy(col_hbm.at[wid], coli)
    base = wid * EW
    for k in range(K):
        idr = rowi.at[pl.ds(k * CH, CH)]
        idc = coli.at[pl.ds(k * CH, CH)]
        cg = pltpu.async_copy(tbl_hbm.at[idr], bufG, sem)
        cb = pltpu.async_copy(pos16_hbm.at[idc], bufB, sem)
        cg.wait()
        cb.wait()
        pltpu.sync_copy(bufB, pc_hbm.at[pl.ds(base + k * CH, CH)])
        pltpu.sync_copy(bufG, g_hbm.at[pl.ds(base + k * CH, CH)])


def _gather(tbl, pos16, row2, col2):
    call = pl.kernel(
        _gather_body,
        out_type=(jax.ShapeDtypeStruct((E2, 16), jnp.float32),
                  jax.ShapeDtypeStruct((E2, G), jnp.float32)),
        mesh=plsc.VectorSubcoreMesh(core_axis_name="c", subcore_axis_name="s"),
        scratch_types=[pltpu.VMEM((EW,), jnp.int32),
                       pltpu.VMEM((EW,), jnp.int32),
                       pltpu.VMEM((CH, G), jnp.float32),
                       pltpu.VMEM((CH, 16), jnp.float32),
                       pltpu.SemaphoreType.DMA],
        compiler_params=pltpu.CompilerParams(use_tc_tiling_on_sc=False),
    )
    return call(tbl, pos16, row2, col2)


BE = 2048


def _edge_body(vals_ref, pc_ref, g_ref, ws1_ref, ws2_ref, wp1_ref,
               wp2_ref, wd1_ref, wd2_ref, out_ref):
    gt = g_ref[...].T
    pct = pc_ref[...].T
    vx = gt[40:41] - pct[0:1]
    vy = gt[41:42] - pct[1:2]
    vz = gt[42:43] - pct[2:3]
    r2 = vx * vx + vy * vy + vz * vz + 1e-12
    length = jnp.sqrt(r2)
    inv = 1.0 / length
    ux = vx * inv
    uy = vy * inv
    uz = vz * inv
    s1x = C1 * ux
    s1y = C1 * uy
    s1z = C1 * uz
    s20 = C2 * ux * uy
    s21 = C2 * uy * uz
    s22 = C3 * (3.0 * uz * uz - 1.0)
    s23 = C2 * ux * uz
    s24 = C4 * (ux * ux - uy * uy)
    valsv = vals_ref[0:1, 0:10].T
    diff = (length - valsv) * vals_ref[0:1, 10:11]
    ap = diff + 1.0
    bp = 1.0 - diff
    sa = jnp.where(ap > 0.0, jnp.exp(-1.0 / jnp.where(ap > 0.0, ap, 1.0)), 0.0)
    sb = jnp.where(bp > 0.0, jnp.exp(-1.0 / jnp.where(bp > 0.0, bp, 1.0)), 0.0)
    emb = EMBS * sa * sb

    def fc(w1t_ref, w2t_ref, oscale):
        h = jnp.dot(w1t_ref[...], emb, preferred_element_type=jnp.float32) * FS
        h = FH * jnp.maximum(h, 0.0)
        return jnp.dot(w2t_ref[...], h,
                       preferred_element_type=jnp.float32) * (FO * oscale)

    ws = fc(ws1_ref, ws2_ref, SC0)
    wp = fc(wp1_ref, wp2_ref, SC0 / 3.0)
    wd = fc(wd1_ref, wd2_ref, SC0 / 5.0)
    ip0 = gt[1:2] * s1x + gt[2:3] * s1y + gt[3:4] * s1z
    ip1 = gt[4:5] * s1x + gt[5:6] * s1y + gt[6:7] * s1z
    ip2 = gt[7:8] * s1x + gt[8:9] * s1y + gt[9:10] * s1z
    id0 = (gt[10:11] * s20 + gt[11:12] * s21 + gt[12:13] * s22
           + gt[13:14] * s23 + gt[14:15] * s24)
    id1 = (gt[15:16] * s20 + gt[16:17] * s21 + gt[17:18] * s22
           + gt[18:19] * s23 + gt[19:20] * s24)
    id2 = (gt[20:21] * s20 + gt[21:22] * s21 + gt[22:23] * s22
           + gt[23:24] * s23 + gt[24:25] * s24)
    id3 = (gt[25:26] * s20 + gt[26:27] * s21 + gt[27:28] * s22
           + gt[28:29] * s23 + gt[29:30] * s24)
    id4 = (gt[30:31] * s20 + gt[31:32] * s21 + gt[32:33] * s22
           + gt[33:34] * s23 + gt[34:35] * s24)
    out_s = gt[0:1] * ws
    out_p = ip0 * wp[0:4] + ip1 * wp[4:8] + ip2 * wp[8:12]
    out_d = (id0 * wd[0:4] + id1 * wd[4:8] + id2 * wd[8:12]
             + id3 * wd[12:16] + id4 * wd[16:20])
    out_t = jnp.concatenate(
        [out_s, out_p, out_d, jnp.zeros_like(out_s)], axis=0)
    out_ref[...] = out_t.T


def _edge(vals, pc, ge, Ws1, Ws2, Wp1, Wp2, Wd1, Wd2):
    nb = E2 // BE
    wspec = lambda shape: pl.BlockSpec(shape, lambda i: (0, 0))
    return pl.pallas_call(
        _edge_body,
        grid=(nb,),
        in_specs=[wspec((1, 16)),
                  pl.BlockSpec((BE, 16), lambda i: (i, 0)),
                  pl.BlockSpec((BE, G), lambda i: (i, 0)),
                  wspec((64, 10)), wspec((4, 64)),
                  wspec((64, 10)), wspec((12, 64)),
                  wspec((64, 10)), wspec((20, 64))],
        out_specs=pl.BlockSpec((BE, SROW), lambda i: (i, 0)),
        out_shape=jax.ShapeDtypeStruct((E2, SROW), jnp.float32),
    )(vals, pc, ge, Ws1, Ws2, Wp1, Wp2, Wd1, Wd2)


def _scatter_body(s_hbm, col_hbm, z_hbm, part_hbm, coli, stage, acc):
    cid = lax.axis_index("c")
    sid = lax.axis_index("s")
    wid = sid * 2 + cid
    pltpu.sync_copy(col_hbm.at[wid], coli)
    pltpu.sync_copy(z_hbm, acc.at[pl.ds(sid * NROWS, NROWS)])
    plsc.subcore_barrier()
    base = wid * EW
    for k in range(K):
        pltpu.sync_copy(s_hbm.at[pl.ds(base + k * CH, CH)], stage)
        pltpu.sync_copy(stage, acc.at[coli.at[k]], add=True)
    plsc.subcore_barrier()
    pltpu.sync_copy(acc.at[pl.ds(sid * NROWS, NROWS)],
                    part_hbm.at[cid, pl.ds(sid * NROWS, NROWS)])


def _scatter(s16, col3, zer):
    call = pl.kernel(
        _scatter_body,
        out_type=jax.ShapeDtypeStruct((2, N, SROW), jnp.float32),
        mesh=plsc.VectorSubcoreMesh(core_axis_name="c", subcore_axis_name="s"),
        scratch_types=[pltpu.VMEM((K, CH), jnp.int32),
                       pltpu.VMEM((CH, SROW), jnp.float32),
                       pltpu.VMEM_SHARED((N, SROW), jnp.float32)],
        compiler_params=pltpu.CompilerParams(use_tc_tiling_on_sc=False),
    )
    return call(s16, col3, zer)


def _combine_body(p_ref, o_ref):
    s = p_ref[0] + p_ref[1]
    o_ref[...] = s[:, 0:12]


def _combine(part):
    nb = 10
    bs = N // nb
    return pl.pallas_call(
        _combine_body,
        grid=(nb,),
        in_specs=[pl.BlockSpec((2, bs, SROW), lambda i: (0, i, 0))],
        out_specs=pl.BlockSpec((bs, 12), lambda i: (i, 0)),
        out_shape=jax.ShapeDtypeStruct((N, 12), jnp.float32),
    )(part)


def kernel(x, edge_index, pos, max_radius, num_nodes, Ws1, Ws2, Wp1, Wp2,
           Wd1, Wd2):
    padz = jnp.zeros((E2 - E,), edge_index.dtype)
    rowp = jnp.concatenate([edge_index[0], padz])
    colp = jnp.concatenate([edge_index[1], padz])
    row2 = rowp.reshape(NW, EW)
    col2 = colp.reshape(NW, EW)
    col3 = colp.reshape(NW, K, CH)
    pos16 = jnp.pad(pos, ((0, 0), (0, 13)))
    mr = jnp.asarray(max_radius, jnp.float32)
    idx = jnp.arange(1, 11, dtype=jnp.float32)
    vals = jnp.concatenate([idx * (mr / 11.0), (11.0 / mr)[None],
                            jnp.zeros((5,), jnp.float32)]).reshape(1, 16)
    tbl = _build_tbl(x, pos, jnp.asarray(_SEL))
    pc, ge = _gather(tbl, pos16, row2, col2)
    s16 = _edge(vals, pc, ge, Ws1.T, Ws2.T, Wp1.T, Wp2.T, Wd1.T, Wd2.T)
    zer = jnp.zeros((NROWS, SROW), jnp.float32)
    part = _scatter(s16, col3, zer)
    return _combine(part)


# speedup vs baseline: 10.5576x; 1.1232x over previous
import functools

import numpy as np
import jax
import jax.numpy as jnp
from jax import lax
from jax.experimental import pallas as pl
from jax.experimental.pallas import tpu as pltpu
from jax.experimental.pallas import tpu_sc as plsc

N = 10000
E = 320000
E2 = 327680
NW = 32
EW = E2 // NW
CH = 1024
K = EW // CH
G = 48
SROW = 16
NSUB = 16
NROWS = N // NSUB

C1 = float(np.sqrt(3.0))
C2 = float(np.sqrt(15.0))
C3 = float(np.sqrt(5.0) / 2.0)
C4 = float(np.sqrt(15.0) / 2.0)
EMBS = float(1.14136 * np.exp(2.0) * np.sqrt(10.0))
FS = float(1.0 / np.sqrt(10.0))
FH = float(np.sqrt(2.0))
FO = 1.0 / 8.0
SC0 = float(1.0 / np.sqrt(32.0))


def _sel_matrix():
    S = np.zeros((162, 40), np.float32)
    k = 0
    S[0, k] = 1.0
    S[1, k] = 1.0
    k += 1
    for i in range(1, 4):
        for j in range(1, 4):
            c = (i * 9 + j) * 2
            S[c, k] = 1.0
            S[c + 1, k] = 1.0
            k += 1
    for i in range(4, 9):
        for j in range(4, 9):
            c = (i * 9 + j) * 2
            S[c, k] = 1.0
            S[c + 1, k] = 1.0
            k += 1
    return S


_SEL = _sel_matrix()


def _prep_body(x_ref, pos_ref, sel_ref, tbl_ref):
    t = jnp.dot(x_ref[...], sel_ref[...], preferred_element_type=jnp.float32)
    tbl_ref[...] = jnp.concatenate(
        [t, pos_ref[...], jnp.zeros((t.shape[0], 5), jnp.float32)], axis=1)


def _build_tbl(x, pos, sel):
    nb = 10
    bs = N // nb
    return pl.pallas_call(
        _prep_body,
        grid=(nb,),
        in_specs=[pl.BlockSpec((bs, 162), lambda i: (i, 0)),
                  pl.BlockSpec((bs, 3), lambda i: (i, 0)),
                  pl.BlockSpec((162, 40), lambda i: (0, 0))],
        out_specs=pl.BlockSpec((bs, G), lambda i: (i, 0)),
        out_shape=jax.ShapeDtypeStruct((N, G), jnp.float32),
    )(x, pos, sel)


def _gather_body(tbl_hbm, posc_hbm, row_hbm, col_hbm, g_hbm,
                 rowi, coli, bufG, bufB, sem):
    cid = lax.axis_index("c")
    sid = lax.axis_index("s")
    wid = sid * 2 + cid
    pltpu.sync_copy(row_hbm.at[wid], rowi)
    pltpu.sync_copy(col_hbm.at[wid], coli)
    base = wid * EW
    for k in range(K):
        idr = rowi.at[pl.ds(k * CH, CH)]
        idc = coli.at[pl.ds(k * CH, CH)]
        cg = pltpu.async_copy(tbl_hbm.at[idr], bufG, sem)
        cb = pltpu.async_copy(posc_hbm.at[idc], bufB, sem)
        cg.wait()
        cb.wait()

        def body(i, carry):
            a = bufG[i, pl.ds(32, 16)]
            b = bufB[i, :]
            bufG[i, pl.ds(32, 16)] = a - b
            return carry

        lax.fori_loop(0, CH, body, 0)
        pltpu.sync_copy(bufG, g_hbm.at[pl.ds(base + k * CH, CH)])


def _gather(tbl, posc, row2, col2):
    call = pl.kernel(
        _gather_body,
        out_type=jax.ShapeDtypeStruct((E2, G), jnp.float32),
        mesh=plsc.VectorSubcoreMesh(core_axis_name="c", subcore_axis_name="s"),
        scratch_types=[pltpu.VMEM((EW,), jnp.int32),
                       pltpu.VMEM((EW,), jnp.int32),
                       pltpu.VMEM((CH, G), jnp.float32),
                       pltpu.VMEM((CH, 16), jnp.float32),
                       pltpu.SemaphoreType.DMA],
        compiler_params=pltpu.CompilerParams(use_tc_tiling_on_sc=False),
    )
    return call(tbl, posc, row2, col2)


BE = 2048


def _edge_body(vals_ref, g_ref, ws1_ref, ws2_ref, wp1_ref,
               wp2_ref, wd1_ref, wd2_ref, out_ref):
    gt = g_ref[...].T
    vx = gt[40:41]
    vy = gt[41:42]
    vz = gt[42:43]
    r2 = vx * vx + vy * vy + vz * vz + 1e-12
    length = jnp.sqrt(r2)
    inv = 1.0 / length
    ux = vx * inv
    uy = vy * inv
    uz = vz * inv
    s1x = C1 * ux
    s1y = C1 * uy
    s1z = C1 * uz
    s20 = C2 * ux * uy
    s21 = C2 * uy * uz
    s22 = C3 * (3.0 * uz * uz - 1.0)
    s23 = C2 * ux * uz
    s24 = C4 * (ux * ux - uy * uy)
    valsv = vals_ref[0:1, 0:10].T
    diff = (length - valsv) * vals_ref[0:1, 10:11]
    ap = diff + 1.0
    bp = 1.0 - diff
    sa = jnp.where(ap > 0.0, jnp.exp(-1.0 / jnp.where(ap > 0.0, ap, 1.0)), 0.0)
    sb = jnp.where(bp > 0.0, jnp.exp(-1.0 / jnp.where(bp > 0.0, bp, 1.0)), 0.0)
    emb = EMBS * sa * sb

    def fc(w1t_ref, w2t_ref, oscale):
        h = jnp.dot(w1t_ref[...], emb, preferred_element_type=jnp.float32) * FS
        h = FH * jnp.maximum(h, 0.0)
        return jnp.dot(w2t_ref[...], h,
                       preferred_element_type=jnp.float32) * (FO * oscale)

    ws = fc(ws1_ref, ws2_ref, SC0)
    wp = fc(wp1_ref, wp2_ref, SC0 / 3.0)
    wd = fc(wd1_ref, wd2_ref, SC0 / 5.0)
    ip0 = gt[1:2] * s1x + gt[2:3] * s1y + gt[3:4] * s1z
    ip1 = gt[4:5] * s1x + gt[5:6] * s1y + gt[6:7] * s1z
    ip2 = gt[7:8] * s1x + gt[8:9] * s1y + gt[9:10] * s1z
    id0 = (gt[10:11] * s20 + gt[11:12] * s21 + gt[12:13] * s22
           + gt[13:14] * s23 + gt[14:15] * s24)
    id1 = (gt[15:16] * s20 + gt[16:17] * s21 + gt[17:18] * s22
           + gt[18:19] * s23 + gt[19:20] * s24)
    id2 = (gt[20:21] * s20 + gt[21:22] * s21 + gt[22:23] * s22
           + gt[23:24] * s23 + gt[24:25] * s24)
    id3 = (gt[25:26] * s20 + gt[26:27] * s21 + gt[27:28] * s22
           + gt[28:29] * s23 + gt[29:30] * s24)
    id4 = (gt[30:31] * s20 + gt[31:32] * s21 + gt[32:33] * s22
           + gt[33:34] * s23 + gt[34:35] * s24)
    out_s = gt[0:1] * ws
    out_p = ip0 * wp[0:4] + ip1 * wp[4:8] + ip2 * wp[8:12]
    out_d = (id0 * wd[0:4] + id1 * wd[4:8] + id2 * wd[8:12]
             + id3 * wd[12:16] + id4 * wd[16:20])
    out_t = jnp.concatenate(
        [out_s, out_p, out_d, jnp.zeros_like(out_s)], axis=0)
    out_ref[...] = out_t.T


def _edge(vals, ge, Ws1, Ws2, Wp1, Wp2, Wd1, Wd2):
    nb = E2 // BE
    wspec = lambda shape: pl.BlockSpec(shape, lambda i: (0, 0))
    return pl.pallas_call(
        _edge_body,
        grid=(nb,),
        in_specs=[wspec((1, 16)),
                  pl.BlockSpec((BE, G), lambda i: (i, 0)),
                  wspec((64, 10)), wspec((4, 64)),
                  wspec((64, 10)), wspec((12, 64)),
                  wspec((64, 10)), wspec((20, 64))],
        out_specs=pl.BlockSpec((BE, SROW), lambda i: (i, 0)),
        out_shape=jax.ShapeDtypeStruct((E2, SROW), jnp.float32),
    )(vals, ge, Ws1, Ws2, Wp1, Wp2, Wd1, Wd2)


def _scatter_body(s_hbm, col_hbm, z_hbm, part_hbm, coli, stage, acc):
    cid = lax.axis_index("c")
    sid = lax.axis_index("s")
    wid = sid * 2 + cid
    pltpu.sync_copy(col_hbm.at[wid], coli)
    pltpu.sync_copy(z_hbm, acc.at[pl.ds(sid * NROWS, NROWS)])
    plsc.subcore_barrier()
    base = wid * EW
    for k in range(K):
        pltpu.sync_copy(s_hbm.at[pl.ds(base + k * CH, CH)], stage)
        pltpu.sync_copy(stage, acc.at[coli.at[k]], add=True)
    plsc.subcore_barrier()
    pltpu.sync_copy(acc.at[pl.ds(sid * NROWS, NROWS)],
                    part_hbm.at[cid, pl.ds(sid * NROWS, NROWS)])


def _scatter(s16, col3, zer):
    call = pl.kernel(
        _scatter_body,
        out_type=jax.ShapeDtypeStruct((2, N, SROW), jnp.float32),
        mesh=plsc.VectorSubcoreMesh(core_axis_name="c", subcore_axis_name="s"),
        scratch_types=[pltpu.VMEM((K, CH), jnp.int32),
                       pltpu.VMEM((CH, SROW), jnp.float32),
                       pltpu.VMEM_SHARED((N, SROW), jnp.float32)],
        compiler_params=pltpu.CompilerParams(use_tc_tiling_on_sc=False),
    )
    return call(s16, col3, zer)


def _combine_body(p_ref, o_ref):
    s = p_ref[0] + p_ref[1]
    o_ref[...] = s[:, 0:12]


def _combine(part):
    nb = 10
    bs = N // nb
    return pl.pallas_call(
        _combine_body,
        grid=(nb,),
        in_specs=[pl.BlockSpec((2, bs, SROW), lambda i: (0, i, 0))],
        out_specs=pl.BlockSpec((bs, 12), lambda i: (i, 0)),
        out_shape=jax.ShapeDtypeStruct((N, 12), jnp.float32),
    )(part)


def kernel(x, edge_index, pos, max_radius, num_nodes, Ws1, Ws2, Wp1, Wp2,
           Wd1, Wd2):
    padz = jnp.zeros((E2 - E,), edge_index.dtype)
    rowp = jnp.concatenate([edge_index[0], padz])
    colp = jnp.concatenate([edge_index[1], padz])
    row2 = rowp.reshape(NW, EW)
    col2 = colp.reshape(NW, EW)
    col3 = colp.reshape(NW, K, CH)
    posc = jnp.pad(pos, ((0, 0), (8, 5)))
    mr = jnp.asarray(max_radius, jnp.float32)
    idx = jnp.arange(1, 11, dtype=jnp.float32)
    vals = jnp.concatenate([idx * (mr / 11.0), (11.0 / mr)[None],
                            jnp.zeros((5,), jnp.float32)]).reshape(1, 16)
    tbl = _build_tbl(x, pos, jnp.asarray(_SEL))
    ge = _gather(tbl, posc, row2, col2)
    s16 = _edge(vals, ge, Ws1.T, Ws2.T, Wp1.T, Wp2.T, Wd1.T, Wd2.T)
    zer = jnp.zeros((NROWS, SROW), jnp.float32)
    part = _scatter(s16, col3, zer)
    return _combine(part)


# speedup vs baseline: 11.2297x; 1.0637x over previous
import functools

import numpy as np
import jax
import jax.numpy as jnp
from jax import lax
from jax.experimental import pallas as pl
from jax.experimental.pallas import tpu as pltpu
from jax.experimental.pallas import tpu_sc as plsc

N = 10000
E = 320000
E2 = 327680
NW = 32
EW = E2 // NW
CH = 1024
K = EW // CH
CHG = 512
PW = 2 * EW
CPAIR = PW // CHG
C0 = 14
G = 48
SROW = 16
NSUB = 16
NROWS = N // NSUB

C1 = float(np.sqrt(3.0))
C2 = float(np.sqrt(15.0))
C3 = float(np.sqrt(5.0) / 2.0)
C4 = float(np.sqrt(15.0) / 2.0)
EMBS = float(1.14136 * np.exp(2.0) * np.sqrt(10.0))
FS = float(1.0 / np.sqrt(10.0))
FH = float(np.sqrt(2.0))
FO = 1.0 / 8.0
SC0 = float(1.0 / np.sqrt(32.0))


def _sel_matrix():
    S = np.zeros((162, 40), np.float32)
    k = 0
    S[0, k] = 1.0
    S[1, k] = 1.0
    k += 1
    for i in range(1, 4):
        for j in range(1, 4):
            c = (i * 9 + j) * 2
            S[c, k] = 1.0
            S[c + 1, k] = 1.0
            k += 1
    for i in range(4, 9):
        for j in range(4, 9):
            c = (i * 9 + j) * 2
            S[c, k] = 1.0
            S[c + 1, k] = 1.0
            k += 1
    return S


_SEL = _sel_matrix()


def _prep_body(x_ref, pos_ref, sel_ref, tbl_ref):
    t = jnp.dot(x_ref[...], sel_ref[...], preferred_element_type=jnp.float32)
    tbl_ref[...] = jnp.concatenate(
        [t, pos_ref[...], jnp.zeros((t.shape[0], 5), jnp.float32)], axis=1)


def _build_tbl(x, pos, sel):
    nb = 10
    bs = N // nb
    return pl.pallas_call(
        _prep_body,
        grid=(nb,),
        in_specs=[pl.BlockSpec((bs, 162), lambda i: (i, 0)),
                  pl.BlockSpec((bs, 3), lambda i: (i, 0)),
                  pl.BlockSpec((162, 40), lambda i: (0, 0))],
        out_specs=pl.BlockSpec((bs, G), lambda i: (i, 0)),
        out_shape=jax.ShapeDtypeStruct((N, G), jnp.float32),
    )(x, pos, sel)


def _gather_body(tbl_hbm, posc_hbm, row_hbm, col_hbm, g_hbm,
                 rowi, coli, bG0, bB0, bG1, bB1, sem0, sem1):
    cid = lax.axis_index("c")
    sid = lax.axis_index("s")
    pltpu.sync_copy(row_hbm.at[sid], rowi)
    pltpu.sync_copy(col_hbm.at[sid], coli)
    pairbase = sid * PW

    def run(first_chunk, nchunks):
        bufs = ((bG0, bB0, sem0), (bG1, bB1, sem1))

        def fire(k, bset):
            off = (first_chunk + k) * CHG
            bG, bB, sem = bset
            return (pltpu.async_copy(tbl_hbm.at[rowi.at[pl.ds(off, CHG)]],
                                     bG, sem),
                    pltpu.async_copy(posc_hbm.at[coli.at[pl.ds(off, CHG)]],
                                     bB, sem))

        pend = [fire(0, bufs[0]), None]
        for k in range(nchunks):
            cur = k % 2
            if k + 1 < nchunks:
                pend[1 - cur] = fire(k + 1, bufs[1 - cur])
            ca, cb = pend[cur]
            ca.wait()
            cb.wait()
            bG, bB, _ = bufs[cur]

            def body(i, carry):
                bG[i, pl.ds(32, 16)] = bG[i, pl.ds(32, 16)] - bB[i, :]
                return carry

            lax.fori_loop(0, CHG, body, 0)
            pltpu.sync_copy(
                bG, g_hbm.at[pl.ds(pairbase + (first_chunk + k) * CHG, CHG)])

    @pl.when(cid == 0)
    def _():
        run(0, C0)

    @pl.when(cid == 1)
    def _():
        run(C0, CPAIR - C0)


def _gather(tbl, posc, row2, col2):
    call = pl.kernel(
        _gather_body,
        out_type=jax.ShapeDtypeStruct((E2, G), jnp.float32),
        mesh=plsc.VectorSubcoreMesh(core_axis_name="c", subcore_axis_name="s"),
        scratch_types=[pltpu.VMEM((PW,), jnp.int32),
                       pltpu.VMEM((PW,), jnp.int32),
                       pltpu.VMEM((CHG, G), jnp.float32),
                       pltpu.VMEM((CHG, 16), jnp.float32),
                       pltpu.VMEM((CHG, G), jnp.float32),
                       pltpu.VMEM((CHG, 16), jnp.float32),
                       pltpu.SemaphoreType.DMA,
                       pltpu.SemaphoreType.DMA],
        compiler_params=pltpu.CompilerParams(use_tc_tiling_on_sc=False),
    )
    return call(tbl, posc, row2, col2)


BE = 2048


def _edge_body(vals_ref, g_ref, ws1_ref, ws2_ref, wp1_ref,
               wp2_ref, wd1_ref, wd2_ref, out_ref):
    gt = g_ref[...].T
    vx = gt[40:41]
    vy = gt[41:42]
    vz = gt[42:43]
    r2 = vx * vx + vy * vy + vz * vz + 1e-12
    length = jnp.sqrt(r2)
    inv = 1.0 / length
    ux = vx * inv
    uy = vy * inv
    uz = vz * inv
    s1x = C1 * ux
    s1y = C1 * uy
    s1z = C1 * uz
    s20 = C2 * ux * uy
    s21 = C2 * uy * uz
    s22 = C3 * (3.0 * uz * uz - 1.0)
    s23 = C2 * ux * uz
    s24 = C4 * (ux * ux - uy * uy)
    valsv = vals_ref[0:1, 0:10].T
    diff = (length - valsv) * vals_ref[0:1, 10:11]
    ap = diff + 1.0
    bp = 1.0 - diff
    sa = jnp.where(ap > 0.0, jnp.exp(-1.0 / jnp.where(ap > 0.0, ap, 1.0)), 0.0)
    sb = jnp.where(bp > 0.0, jnp.exp(-1.0 / jnp.where(bp > 0.0, bp, 1.0)), 0.0)
    emb = EMBS * sa * sb

    def fc(w1t_ref, w2t_ref, oscale):
        h = jnp.dot(w1t_ref[...], emb, preferred_element_type=jnp.float32) * FS
        h = FH * jnp.maximum(h, 0.0)
        return jnp.dot(w2t_ref[...], h,
                       preferred_element_type=jnp.float32) * (FO * oscale)

    ws = fc(ws1_ref, ws2_ref, SC0)
    wp = fc(wp1_ref, wp2_ref, SC0 / 3.0)
    wd = fc(wd1_ref, wd2_ref, SC0 / 5.0)
    ip0 = gt[1:2] * s1x + gt[2:3] * s1y + gt[3:4] * s1z
    ip1 = gt[4:5] * s1x + gt[5:6] * s1y + gt[6:7] * s1z
    ip2 = gt[7:8] * s1x + gt[8:9] * s1y + gt[9:10] * s1z
    id0 = (gt[10:11] * s20 + gt[11:12] * s21 + gt[12:13] * s22
           + gt[13:14] * s23 + gt[14:15] * s24)
    id1 = (gt[15:16] * s20 + gt[16:17] * s21 + gt[17:18] * s22
           + gt[18:19] * s23 + gt[19:20] * s24)
    id2 = (gt[20:21] * s20 + gt[21:22] * s21 + gt[22:23] * s22
           + gt[23:24] * s23 + gt[24:25] * s24)
    id3 = (gt[25:26] * s20 + gt[26:27] * s21 + gt[27:28] * s22
           + gt[28:29] * s23 + gt[29:30] * s24)
    id4 = (gt[30:31] * s20 + gt[31:32] * s21 + gt[32:33] * s22
           + gt[33:34] * s23 + gt[34:35] * s24)
    out_s = gt[0:1] * ws
    out_p = ip0 * wp[0:4] + ip1 * wp[4:8] + ip2 * wp[8:12]
    out_d = (id0 * wd[0:4] + id1 * wd[4:8] + id2 * wd[8:12]
             + id3 * wd[12:16] + id4 * wd[16:20])
    out_t = jnp.concatenate(
        [out_s, out_p, out_d, jnp.zeros_like(out_s)], axis=0)
    out_ref[...] = out_t.T


def _edge(vals, ge, Ws1, Ws2, Wp1, Wp2, Wd1, Wd2):
    nb = E2 // BE
    wspec = lambda shape: pl.BlockSpec(shape, lambda i: (0, 0))
    return pl.pallas_call(
        _edge_body,
        grid=(nb,),
        in_specs=[wspec((1, 16)),
                  pl.BlockSpec((BE, G), lambda i: (i, 0)),
                  wspec((64, 10)), wspec((4, 64)),
                  wspec((64, 10)), wspec((12, 64)),
                  wspec((64, 10)), wspec((20, 64))],
        out_specs=pl.BlockSpec((BE, SROW), lambda i: (i, 0)),
        out_shape=jax.ShapeDtypeStruct((E2, SROW), jnp.float32),
    )(vals, ge, Ws1, Ws2, Wp1, Wp2, Wd1, Wd2)


def _scatter_body(s_hbm, col_hbm, z_hbm, part_hbm, coli, stage, acc):
    cid = lax.axis_index("c")
    sid = lax.axis_index("s")
    wid = sid * 2 + cid
    pltpu.sync_copy(col_hbm.at[wid], coli)
    pltpu.sync_copy(z_hbm, acc.at[pl.ds(sid * NROWS, NROWS)])
    plsc.subcore_barrier()
    base = wid * EW
    for k in range(K):
        pltpu.sync_copy(s_hbm.at[pl.ds(base + k * CH, CH)], stage)
        pltpu.sync_copy(stage, acc.at[coli.at[k]], add=True)
    plsc.subcore_barrier()
    pltpu.sync_copy(acc.at[pl.ds(sid * NROWS, NROWS)],
                    part_hbm.at[cid, pl.ds(sid * NROWS, NROWS)])


def _scatter(s16, col3, zer):
    call = pl.kernel(
        _scatter_body,
        out_type=jax.ShapeDtypeStruct((2, N, SROW), jnp.float32),
        mesh=plsc.VectorSubcoreMesh(core_axis_name="c", subcore_axis_name="s"),
        scratch_types=[pltpu.VMEM((K, CH), jnp.int32),
                       pltpu.VMEM((CH, SROW), jnp.float32),
                       pltpu.VMEM_SHARED((N, SROW), jnp.float32)],
        compiler_params=pltpu.CompilerParams(use_tc_tiling_on_sc=False),
    )
    return call(s16, col3, zer)


def _combine_body(p_ref, o_ref):
    s = p_ref[0] + p_ref[1]
    o_ref[...] = s[:, 0:12]


def _combine(part):
    nb = 10
    bs = N // nb
    return pl.pallas_call(
        _combine_body,
        grid=(nb,),
        in_specs=[pl.BlockSpec((2, bs, SROW), lambda i: (0, i, 0))],
        out_specs=pl.BlockSpec((bs, 12), lambda i: (i, 0)),
        out_shape=jax.ShapeDtypeStruct((N, 12), jnp.float32),
    )(part)


def kernel(x, edge_index, pos, max_radius, num_nodes, Ws1, Ws2, Wp1, Wp2,
           Wd1, Wd2):
    padz = jnp.zeros((E2 - E,), edge_index.dtype)
    rowp = jnp.concatenate([edge_index[0], padz])
    colp = jnp.concatenate([edge_index[1], padz])
    row2 = rowp.reshape(NW // 2, PW)
    col2 = colp.reshape(NW // 2, PW)
    col3 = colp.reshape(NW, K, CH)
    posc = jnp.pad(pos, ((0, 0), (8, 5)))
    mr = jnp.asarray(max_radius, jnp.float32)
    idx = jnp.arange(1, 11, dtype=jnp.float32)
    vals = jnp.concatenate([idx * (mr / 11.0), (11.0 / mr)[None],
                            jnp.zeros((5,), jnp.float32)]).reshape(1, 16)
    tbl = _build_tbl(x, pos, jnp.asarray(_SEL))
    ge = _gather(tbl, posc, row2, col2)
    s16 = _edge(vals, ge, Ws1.T, Ws2.T, Wp1.T, Wp2.T, Wd1.T, Wd2.T)
    zer = jnp.zeros((NROWS, SROW), jnp.float32)
    part = _scatter(s16, col3, zer)
    return _combine(part)


# speedup vs baseline: 11.6973x; 1.0416x over previous
import functools

import numpy as np
import jax
import jax.numpy as jnp
from jax import lax
from jax.experimental import pallas as pl
from jax.experimental.pallas import tpu as pltpu
from jax.experimental.pallas import tpu_sc as plsc

N = 10000
E = 320000
E2 = 327680
NW = 32
EW = E2 // NW
CH = 1024
K = EW // CH
CHG = 512
PW = 2 * EW
CPAIR = PW // CHG
C0 = 5
G = 48
SROW = 16
NSUB = 16
NROWS = N // NSUB

C1 = float(np.sqrt(3.0))
C2 = float(np.sqrt(15.0))
C3 = float(np.sqrt(5.0) / 2.0)
C4 = float(np.sqrt(15.0) / 2.0)
EMBS = float(1.14136 * np.exp(2.0) * np.sqrt(10.0))
FS = float(1.0 / np.sqrt(10.0))
FH = float(np.sqrt(2.0))
FO = 1.0 / 8.0
SC0 = float(1.0 / np.sqrt(32.0))


def _sel_matrix():
    S = np.zeros((162, 40), np.float32)
    k = 0
    S[0, k] = 1.0
    S[1, k] = 1.0
    k += 1
    for i in range(1, 4):
        for j in range(1, 4):
            c = (i * 9 + j) * 2
            S[c, k] = 1.0
            S[c + 1, k] = 1.0
            k += 1
    for i in range(4, 9):
        for j in range(4, 9):
            c = (i * 9 + j) * 2
            S[c, k] = 1.0
            S[c + 1, k] = 1.0
            k += 1
    return S


_SEL = _sel_matrix()


def _prep_body(x_ref, pos_ref, sel_ref, tbl_ref):
    t = jnp.dot(x_ref[...], sel_ref[...], preferred_element_type=jnp.float32)
    tbl_ref[...] = jnp.concatenate(
        [t, pos_ref[...], jnp.zeros((t.shape[0], 5), jnp.float32)], axis=1)


def _build_tbl(x, pos, sel):
    nb = 10
    bs = N // nb
    return pl.pallas_call(
        _prep_body,
        grid=(nb,),
        in_specs=[pl.BlockSpec((bs, 162), lambda i: (i, 0)),
                  pl.BlockSpec((bs, 3), lambda i: (i, 0)),
                  pl.BlockSpec((162, 40), lambda i: (0, 0))],
        out_specs=pl.BlockSpec((bs, G), lambda i: (i, 0)),
        out_shape=jax.ShapeDtypeStruct((N, G), jnp.float32),
    )(x, pos, sel)


def _gather_body(tbl_hbm, posc_hbm, row_hbm, col_hbm, g_hbm,
                 rowi, coli, bG0, bB0, bG1, bB1, sem0, sem1):
    cid = lax.axis_index("c")
    sid = lax.axis_index("s")
    pltpu.sync_copy(row_hbm.at[sid], rowi)
    pltpu.sync_copy(col_hbm.at[sid], coli)
    pairbase = sid * PW

    def run(first_chunk, nchunks):
        bufs = ((bG0, bB0, sem0), (bG1, bB1, sem1))

        def fire(k, bset):
            off = (first_chunk + k) * CHG
            bG, bB, sem = bset
            return (pltpu.async_copy(tbl_hbm.at[rowi.at[pl.ds(off, CHG)]],
                                     bG, sem),
                    pltpu.async_copy(posc_hbm.at[coli.at[pl.ds(off, CHG)]],
                                     bB, sem))

        pend = [fire(0, bufs[0]), None]
        for k in range(nchunks):
            cur = k % 2
            if k + 1 < nchunks:
                pend[1 - cur] = fire(k + 1, bufs[1 - cur])
            ca, cb = pend[cur]
            ca.wait()
            cb.wait()
            bG, bB, _ = bufs[cur]

            def body(i, carry):
                bG[i, pl.ds(32, 16)] = bG[i, pl.ds(32, 16)] - bB[i, :]
                return carry

            lax.fori_loop(0, CHG, body, 0)
            pltpu.sync_copy(
                bG, g_hbm.at[pl.ds(pairbase + (first_chunk + k) * CHG, CHG)])

    @pl.when(cid == 0)
    def _():
        run(0, C0)

    @pl.when(cid == 1)
    def _():
        run(C0, CPAIR - C0)


def _gather(tbl, posc, row2, col2):
    call = pl.kernel(
        _gather_body,
        out_type=jax.ShapeDtypeStruct((E2, G), jnp.float32),
        mesh=plsc.VectorSubcoreMesh(core_axis_name="c", subcore_axis_name="s"),
        scratch_types=[pltpu.VMEM((PW,), jnp.int32),
                       pltpu.VMEM((PW,), jnp.int32),
                       pltpu.VMEM((CHG, G), jnp.float32),
                       pltpu.VMEM((CHG, 16), jnp.float32),
                       pltpu.VMEM((CHG, G), jnp.float32),
                       pltpu.VMEM((CHG, 16), jnp.float32),
                       pltpu.SemaphoreType.DMA,
                       pltpu.SemaphoreType.DMA],
        compiler_params=pltpu.CompilerParams(use_tc_tiling_on_sc=False),
    )
    return call(tbl, posc, row2, col2)


BE = 4096


def _edge_body(vals_ref, g_ref, ws1_ref, ws2_ref, wp1_ref,
               wp2_ref, wd1_ref, wd2_ref, out_ref):
    gt = g_ref[...].T
    vx = gt[40:41]
    vy = gt[41:42]
    vz = gt[42:43]
    r2 = vx * vx + vy * vy + vz * vz + 1e-12
    length = jnp.sqrt(r2)
    inv = 1.0 / length
    ux = vx * inv
    uy = vy * inv
    uz = vz * inv
    s1x = C1 * ux
    s1y = C1 * uy
    s1z = C1 * uz
    s20 = C2 * ux * uy
    s21 = C2 * uy * uz
    s22 = C3 * (3.0 * uz * uz - 1.0)
    s23 = C2 * ux * uz
    s24 = C4 * (ux * ux - uy * uy)
    valsv = vals_ref[0:1, 0:10].T
    diff = (length - valsv) * vals_ref[0:1, 10:11]
    ap = diff + 1.0
    bp = 1.0 - diff
    sa = jnp.where(ap > 0.0, jnp.exp(-1.0 / jnp.where(ap > 0.0, ap, 1.0)), 0.0)
    sb = jnp.where(bp > 0.0, jnp.exp(-1.0 / jnp.where(bp > 0.0, bp, 1.0)), 0.0)
    emb = EMBS * sa * sb

    def fc(w1t_ref, w2t_ref, oscale):
        h = jnp.dot(w1t_ref[...], emb, preferred_element_type=jnp.float32) * FS
        h = FH * jnp.maximum(h, 0.0)
        return jnp.dot(w2t_ref[...], h,
                       preferred_element_type=jnp.float32) * (FO * oscale)

    ws = fc(ws1_ref, ws2_ref, SC0)
    wp = fc(wp1_ref, wp2_ref, SC0 / 3.0)
    wd = fc(wd1_ref, wd2_ref, SC0 / 5.0)
    ip0 = gt[1:2] * s1x + gt[2:3] * s1y + gt[3:4] * s1z
    ip1 = gt[4:5] * s1x + gt[5:6] * s1y + gt[6:7] * s1z
    ip2 = gt[7:8] * s1x + gt[8:9] * s1y + gt[9:10] * s1z
    id0 = (gt[10:11] * s20 + gt[11:12] * s21 + gt[12:13] * s22
           + gt[13:14] * s23 + gt[14:15] * s24)
    id1 = (gt[15:16] * s20 + gt[16:17] * s21 + gt[17:18] * s22
           + gt[18:19] * s23 + gt[19:20] * s24)
    id2 = (gt[20:21] * s20 + gt[21:22] * s21 + gt[22:23] * s22
           + gt[23:24] * s23 + gt[24:25] * s24)
    id3 = (gt[25:26] * s20 + gt[26:27] * s21 + gt[27:28] * s22
           + gt[28:29] * s23 + gt[29:30] * s24)
    id4 = (gt[30:31] * s20 + gt[31:32] * s21 + gt[32:33] * s22
           + gt[33:34] * s23 + gt[34:35] * s24)
    out_s = gt[0:1] * ws
    out_p = ip0 * wp[0:4] + ip1 * wp[4:8] + ip2 * wp[8:12]
    out_d = (id0 * wd[0:4] + id1 * wd[4:8] + id2 * wd[8:12]
             + id3 * wd[12:16] + id4 * wd[16:20])
    out_t = jnp.concatenate(
        [out_s, out_p, out_d, jnp.zeros_like(out_s)], axis=0)
    out_ref[...] = out_t.T


def _edge(vals, ge, Ws1, Ws2, Wp1, Wp2, Wd1, Wd2):
    nb = E2 // BE
    wspec = lambda shape: pl.BlockSpec(shape, lambda i: (0, 0))
    return pl.pallas_call(
        _edge_body,
        grid=(nb,),
        in_specs=[wspec((1, 16)),
                  pl.BlockSpec((BE, G), lambda i: (i, 0)),
                  wspec((64, 10)), wspec((4, 64)),
                  wspec((64, 10)), wspec((12, 64)),
                  wspec((64, 10)), wspec((20, 64))],
        out_specs=pl.BlockSpec((BE, SROW), lambda i: (i, 0)),
        out_shape=jax.ShapeDtypeStruct((E2, SROW), jnp.float32),
    )(vals, ge, Ws1, Ws2, Wp1, Wp2, Wd1, Wd2)


def _scatter_body(s_hbm, col_hbm, z_hbm, part_hbm, coli, stage, acc):
    cid = lax.axis_index("c")
    sid = lax.axis_index("s")
    wid = sid * 2 + cid
    pltpu.sync_copy(col_hbm.at[wid], coli)
    pltpu.sync_copy(z_hbm, acc.at[pl.ds(sid * NROWS, NROWS)])
    plsc.subcore_barrier()
    base = wid * EW
    for k in range(K):
        pltpu.sync_copy(s_hbm.at[pl.ds(base + k * CH, CH)], stage)
        pltpu.sync_copy(stage, acc.at[coli.at[k]], add=True)
    plsc.subcore_barrier()
    pltpu.sync_copy(acc.at[pl.ds(sid * NROWS, NROWS)],
                    part_hbm.at[cid, pl.ds(sid * NROWS, NROWS)])


def _scatter(s16, col3, zer):
    call = pl.kernel(
        _scatter_body,
        out_type=jax.ShapeDtypeStruct((2, N, SROW), jnp.float32),
        mesh=plsc.VectorSubcoreMesh(core_axis_name="c", subcore_axis_name="s"),
        scratch_types=[pltpu.VMEM((K, CH), jnp.int32),
                       pltpu.VMEM((CH, SROW), jnp.float32),
                       pltpu.VMEM_SHARED((N, SROW), jnp.float32)],
        compiler_params=pltpu.CompilerParams(use_tc_tiling_on_sc=False),
    )
    return call(s16, col3, zer)


def _combine_body(p_ref, o_ref):
    s = p_ref[0] + p_ref[1]
    o_ref[...] = s[:, 0:12]


def _combine(part):
    nb = 10
    bs = N // nb
    return pl.pallas_call(
        _combine_body,
        grid=(nb,),
        in_specs=[pl.BlockSpec((2, bs, SROW), lambda i: (0, i, 0))],
        out_specs=pl.BlockSpec((bs, 12), lambda i: (i, 0)),
        out_shape=jax.ShapeDtypeStruct((N, 12), jnp.float32),
    )(part)


def kernel(x, edge_index, pos, max_radius, num_nodes, Ws1, Ws2, Wp1, Wp2,
           Wd1, Wd2):
    padz = jnp.zeros((E2 - E,), edge_index.dtype)
    rowp = jnp.concatenate([edge_index[0], padz])
    colp = jnp.concatenate([edge_index[1], padz])
    row2 = rowp.reshape(NW // 2, PW)
    col2 = colp.reshape(NW // 2, PW)
    col3 = colp.reshape(NW, K, CH)
    posc = jnp.pad(pos, ((0, 0), (8, 5)))
    mr = jnp.asarray(max_radius, jnp.float32)
    idx = jnp.arange(1, 11, dtype=jnp.float32)
    vals = jnp.concatenate([idx * (mr / 11.0), (11.0 / mr)[None],
                            jnp.zeros((5,), jnp.float32)]).reshape(1, 16)
    tbl = _build_tbl(x, pos, jnp.asarray(_SEL))
    ge = _gather(tbl, posc, row2, col2)
    s16 = _edge(vals, ge, Ws1.T, Ws2.T, Wp1.T, Wp2.T, Wd1.T, Wd2.T)
    zer = jnp.zeros((NROWS, SROW), jnp.float32)
    part = _scatter(s16, col3, zer)
    return _combine(part)
